# baseline (speedup 1.0000x reference)
import functools

import jax
import jax.numpy as jnp
from jax.experimental import pallas as pl

N = 10000
IN_DIM = 64
HID = 64
K = 2
OUT = 64
SUPPORTS_LEN = 2
NUM_MAT = SUPPORTS_LEN * K + 1

ROW_BLK = 2000


def _mm_body(x_ref, w_ref, o_ref):
    o_ref[...] = jnp.dot(x_ref[...], w_ref[...], preferred_element_type=jnp.float32)


def _pallas_matmul(x, w):
    m, k = x.shape
    n = w.shape[1]
    grid = (m // ROW_BLK,)
    return pl.pallas_call(
        _mm_body,
        grid=grid,
        in_specs=[
            pl.BlockSpec((ROW_BLK, k), lambda i: (i, 0)),
            pl.BlockSpec((k, n), lambda i: (0, 0)),
        ],
        out_specs=pl.BlockSpec((ROW_BLK, n), lambda i: (i, 0)),
        out_shape=jax.ShapeDtypeStruct((m, n), jnp.float32),
    )(x, w)


def _spmm(idx, vals, x):
    rows = idx[0]
    cols = idx[1]
    return jax.ops.segment_sum(x[cols] * vals[:, None], rows, num_segments=N)


def kernel(inputs, state, support0_indices, support0_values, support1_indices,
           support1_values, weight, biases, output_size):
    batch = inputs.shape[0]
    xin = inputs.reshape(batch, N, -1)
    st = state.reshape(batch, N, -1)
    xs = jnp.concatenate([xin, st], axis=2)
    D = xs.shape[2]
    x0 = jnp.transpose(xs, (1, 2, 0)).reshape(N, D * batch)

    p1 = _spmm(support0_indices, support0_values, x0)
    p2 = _spmm(support0_indices, support0_values, p1)
    p3 = _spmm(support1_indices, support1_values, p1)
    p4 = _spmm(support1_indices, support1_values, p3)

    w = weight.reshape(D, NUM_MAT, OUT)
    w_eff = jnp.stack([
        w[:, 0] - w[:, 2],
        w[:, 1] - w[:, 4],
        2.0 * w[:, 2],
        w[:, 3],
        2.0 * w[:, 4],
    ], axis=1).reshape(D * NUM_MAT, OUT)

    mats = jnp.stack([x0, p1, p2, p3, p4], axis=0)
    mats = mats.reshape(NUM_MAT, N, D, batch)
    lhs = jnp.transpose(mats, (3, 1, 2, 0)).reshape(batch * N, D * NUM_MAT)

    x = _pallas_matmul(lhs, w_eff)
    x = x + biases
    out_dim = weight.shape[1]
    x = x + (jnp.asarray(output_size, dtype=x.dtype) - out_dim)
    return x.reshape(batch, N * out_dim)


# speedup vs baseline: 1.8883x; 1.8883x over previous
import functools

import jax
import jax.numpy as jnp
from jax import lax
from jax.experimental import pallas as pl
from jax.experimental.pallas import tpu as pltpu
from jax.experimental.pallas import tpu_sc as plsc

N = 10000
NP = 10240
D = 128
OUT = 64
B = 8
E = 160000
NUM_MAT = 5

NC = 2
NS = 16
LANES = 16

EPT = 10112
NBLK = EPT // 128
RPT = NP // NS
B_PER_SC = B // NC

ROW_BLK = 2048


def _spmm_stage_body(src, cols, rows, vals, zeros, out,
                     rows_v, gidx_v, vals_v, buf,
                     acc_sh, sem):
    c = lax.axis_index("c")
    s = lax.axis_index("s")

    pltpu.sync_copy(cols.at[s], gidx_v)
    pltpu.sync_copy(rows.at[s], rows_v)
    pltpu.sync_copy(vals.at[pl.ds(s * EPT, EPT)], vals_v)

    for b_i in range(B_PER_SC):
        base = (c * B_PER_SC + b_i) * NP
        delta = c * B_PER_SC * NP if b_i == 0 else NP

        plsc.subcore_barrier()
        pltpu.sync_copy(zeros, acc_sh.at[pl.ds(s * RPT, RPT)])
        plsc.subcore_barrier()

        def _gidx(i, carry):
            for j in range(D // LANES):
                sl = pl.ds(j * LANES, LANES)
                gidx_v[i, sl] = gidx_v[i, sl] + delta
            return carry

        lax.fori_loop(0, NBLK, _gidx, 0)

        def _edge_block(blk, carry):
            pltpu.async_copy(src.at[gidx_v.at[blk]], buf, sem).wait()

            def _scale(e, carry2):
                v16 = plsc.load_gather(vals_v, [jnp.full((LANES,), blk * 128 + e, jnp.int32)])
                for j in range(D // LANES):
                    sl = pl.ds(j * LANES, LANES)
                    buf[e, sl] = buf[e, sl] * v16
                return carry2

            lax.fori_loop(0, 128, _scale, 0)
            pltpu.sync_copy(buf, acc_sh.at[rows_v.at[blk]], add=True)
            return carry

        lax.fori_loop(0, NBLK, _edge_block, 0)

        plsc.subcore_barrier()
        pltpu.sync_copy(acc_sh.at[pl.ds(s * RPT, RPT)],
                        out.at[pl.ds(base + s * RPT, RPT)])


def _spmm_stage(src, cols, rows, vals, zeros):
    mesh = plsc.VectorSubcoreMesh(core_axis_name="c", subcore_axis_name="s")
    return pl.kernel(
        _spmm_stage_body,
        out_type=jax.ShapeDtypeStruct((B * NP, D), jnp.float32),
        mesh=mesh,
        compiler_params=pltpu.CompilerParams(needs_layout_passes=False),
        scratch_types=[
            pltpu.VMEM((NBLK, 128), jnp.int32),
            pltpu.VMEM((NBLK, 128), jnp.int32),
            pltpu.VMEM((EPT,), jnp.float32),
            pltpu.VMEM((128, D), jnp.float32),
            pltpu.VMEM_SHARED((NP, D), jnp.float32),
            pltpu.SemaphoreType.DMA,
        ],
    )(src, cols, rows, vals, zeros)


def _prep_edges(idx, vals):
    pad = NS * EPT - E
    cols = jnp.concatenate([idx[1], jnp.zeros((pad,), jnp.int32)])
    rows = jnp.concatenate([idx[0], jnp.zeros((pad,), jnp.int32)])
    v = jnp.concatenate([vals, jnp.zeros((pad,), jnp.float32)])
    return (cols.reshape(NS, NBLK, 128), rows.reshape(NS, NBLK, 128), v)


def _mm5_body(x0_ref, p1_ref, p2_ref, p3_ref, p4_ref, w_ref, b_ref, o_ref):
    acc = jnp.dot(x0_ref[...], w_ref[0], preferred_element_type=jnp.float32)
    acc += jnp.dot(p1_ref[...], w_ref[1], preferred_element_type=jnp.float32)
    acc += jnp.dot(p2_ref[...], w_ref[2], preferred_element_type=jnp.float32)
    acc += jnp.dot(p3_ref[...], w_ref[3], preferred_element_type=jnp.float32)
    acc += jnp.dot(p4_ref[...], w_ref[4], preferred_element_type=jnp.float32)
    o_ref[...] = acc + b_ref[...]


def _mm5(x0, p1, p2, p3, p4, w_eff, bias_eff):
    m = x0.shape[0]
    grid = (m // ROW_BLK,)
    blk = pl.BlockSpec((ROW_BLK, D), lambda i: (i, 0))
    return pl.pallas_call(
        _mm5_body,
        grid=grid,
        in_specs=[blk, blk, blk, blk, blk,
                  pl.BlockSpec((NUM_MAT, D, OUT), lambda i: (0, 0, 0)),
                  pl.BlockSpec((1, OUT), lambda i: (0, 0))],
        out_specs=pl.BlockSpec((ROW_BLK, OUT), lambda i: (i, 0)),
        out_shape=jax.ShapeDtypeStruct((m, OUT), jnp.float32),
    )(x0, p1, p2, p3, p4, w_eff, bias_eff)


def kernel(inputs, state, support0_indices, support0_values, support1_indices,
           support1_values, weight, biases, output_size):
    batch = inputs.shape[0]
    xin = inputs.reshape(batch, N, -1)
    st = state.reshape(batch, N, -1)
    x0 = jnp.concatenate([xin, st], axis=2)
    x0 = jnp.pad(x0, ((0, 0), (0, NP - N), (0, 0))).reshape(batch * NP, D)

    c0, r0, v0 = _prep_edges(support0_indices, support0_values)
    c1, r1, v1 = _prep_edges(support1_indices, support1_values)

    zeros = jnp.zeros((RPT, D), jnp.float32)
    p1 = _spmm_stage(x0, c0, r0, v0, zeros)
    p2 = _spmm_stage(p1, c0, r0, v0, zeros)
    p3 = _spmm_stage(p1, c1, r1, v1, zeros)
    p4 = _spmm_stage(p3, c1, r1, v1, zeros)

    w = weight.reshape(D, NUM_MAT, OUT)
    w_eff = jnp.stack([
        w[:, 0] - w[:, 2],
        w[:, 1] - w[:, 4],
        2.0 * w[:, 2],
        w[:, 3],
        2.0 * w[:, 4],
    ], axis=0)

    out_dim = weight.shape[1]
    bias_eff = (biases + (jnp.asarray(output_size, jnp.float32) - out_dim)).reshape(1, OUT)

    res = _mm5(x0, p1, p2, p3, p4, w_eff, bias_eff)
    res = res.reshape(batch, NP, out_dim)[:, :N, :]
    return res.reshape(batch, N * out_dim)
